# baseline (speedup 1.0000x reference)
import jax
import jax.numpy as jnp
from jax.experimental import pallas as pl

N = 20000
D = 1024
BN = 1000


def _fused_heads(x_ref, wc_ref, bc_ref, wb_ref, bb_ref, sc_ref, bd_ref):
    x = x_ref[...]
    sc_ref[...] = (
        jnp.dot(x, wc_ref[...], preferred_element_type=jnp.float32) + bc_ref[...]
    )
    bd_ref[...] = (
        jnp.dot(x, wb_ref[...], preferred_element_type=jnp.float32) + bb_ref[...]
    )


def kernel(x, W_cls, b_cls, W_box, b_box):
    n, d = x.shape
    kc = W_cls.shape[1]
    kb = W_box.shape[1]
    bc = b_cls.reshape(1, kc)
    bb = b_box.reshape(1, kb)
    grid = (n // BN,)
    scores, deltas = pl.pallas_call(
        _fused_heads,
        grid=grid,
        in_specs=[
            pl.BlockSpec((BN, d), lambda i: (i, 0)),
            pl.BlockSpec((d, kc), lambda i: (0, 0)),
            pl.BlockSpec((1, kc), lambda i: (0, 0)),
            pl.BlockSpec((d, kb), lambda i: (0, 0)),
            pl.BlockSpec((1, kb), lambda i: (0, 0)),
        ],
        out_specs=[
            pl.BlockSpec((BN, kc), lambda i: (i, 0)),
            pl.BlockSpec((BN, kb), lambda i: (i, 0)),
        ],
        out_shape=[
            jax.ShapeDtypeStruct((n, kc), jnp.float32),
            jax.ShapeDtypeStruct((n, kb), jnp.float32),
        ],
    )(x, W_cls, bc, W_box, bb)
    return (scores, deltas)


# speedup vs baseline: 1.0636x; 1.0636x over previous
import jax
import jax.numpy as jnp
from jax.experimental import pallas as pl

N = 20000
D = 1024
BN = 2000


def _fused_heads(x_ref, wc_ref, bc_ref, wb_ref, bb_ref, sc_ref, bd_ref):
    x = x_ref[...]
    sc_ref[...] = (
        jnp.dot(x, wc_ref[...], preferred_element_type=jnp.float32) + bc_ref[...]
    )
    bd_ref[...] = (
        jnp.dot(x, wb_ref[...], preferred_element_type=jnp.float32) + bb_ref[...]
    )


def kernel(x, W_cls, b_cls, W_box, b_box):
    n, d = x.shape
    kc = W_cls.shape[1]
    kb = W_box.shape[1]
    bc = b_cls.reshape(1, kc)
    bb = b_box.reshape(1, kb)
    grid = (n // BN,)
    scores, deltas = pl.pallas_call(
        _fused_heads,
        grid=grid,
        in_specs=[
            pl.BlockSpec((BN, d), lambda i: (i, 0)),
            pl.BlockSpec((d, kc), lambda i: (0, 0)),
            pl.BlockSpec((1, kc), lambda i: (0, 0)),
            pl.BlockSpec((d, kb), lambda i: (0, 0)),
            pl.BlockSpec((1, kb), lambda i: (0, 0)),
        ],
        out_specs=[
            pl.BlockSpec((BN, kc), lambda i: (i, 0)),
            pl.BlockSpec((BN, kb), lambda i: (i, 0)),
        ],
        out_shape=[
            jax.ShapeDtypeStruct((n, kc), jnp.float32),
            jax.ShapeDtypeStruct((n, kb), jnp.float32),
        ],
    )(x, W_cls, bc, W_box, bb)
    return (scores, deltas)


# speedup vs baseline: 1.1175x; 1.0507x over previous
import jax
import jax.numpy as jnp
from jax import lax
from jax.experimental import pallas as pl
from jax.experimental.pallas import tpu as pltpu

N = 20000
D = 1024
BN = 1000
NBUF = 4
NSTEPS = N // BN


def _fused_heads(x_hbm, wc, bc, wb, bb, sc_hbm, bd_hbm,
                 x_buf, sc_buf, bd_buf, x_sem, sc_sem, bd_sem):
    def x_copy(i, slot):
        return pltpu.make_async_copy(
            x_hbm.at[pl.ds(i * BN, BN), :], x_buf.at[slot], x_sem.at[slot])

    def sc_copy(i, slot):
        return pltpu.make_async_copy(
            sc_buf.at[slot], sc_hbm.at[pl.ds(i * BN, BN), :], sc_sem.at[slot])

    def bd_copy(i, slot):
        return pltpu.make_async_copy(
            bd_buf.at[slot], bd_hbm.at[pl.ds(i * BN, BN), :], bd_sem.at[slot])

    for i in range(NBUF):
        x_copy(i, i).start()

    W_c = wc[...]
    W_b = wb[...]
    b_c = bc[...]
    b_b = bb[...]

    def step(i, carry):
        slot = lax.rem(i, NBUF)
        x_copy(i, slot).wait()

        @pl.when(i >= NBUF)
        def _():
            sc_copy(i - NBUF, slot).wait()
            bd_copy(i - NBUF, slot).wait()

        x = x_buf[slot]
        sc_buf[slot] = jnp.dot(x, W_c, preferred_element_type=jnp.float32) + b_c
        bd_buf[slot] = jnp.dot(x, W_b, preferred_element_type=jnp.float32) + b_b
        sc_copy(i, slot).start()
        bd_copy(i, slot).start()

        @pl.when(i + NBUF < NSTEPS)
        def _():
            x_copy(i + NBUF, slot).start()

        return carry

    lax.fori_loop(0, NSTEPS, step, 0)

    for j in range(NBUF):
        i = NSTEPS - NBUF + j
        sc_copy(i, i % NBUF).wait()
        bd_copy(i, i % NBUF).wait()


def kernel(x, W_cls, b_cls, W_box, b_box):
    n, d = x.shape
    kc = W_cls.shape[1]
    kb = W_box.shape[1]
    bc = b_cls.reshape(1, kc)
    bb = b_box.reshape(1, kb)
    scores, deltas = pl.pallas_call(
        _fused_heads,
        in_specs=[
            pl.BlockSpec(memory_space=pl.ANY),
            pl.BlockSpec(memory_space=pltpu.VMEM),
            pl.BlockSpec(memory_space=pltpu.VMEM),
            pl.BlockSpec(memory_space=pltpu.VMEM),
            pl.BlockSpec(memory_space=pltpu.VMEM),
        ],
        out_specs=[
            pl.BlockSpec(memory_space=pl.ANY),
            pl.BlockSpec(memory_space=pl.ANY),
        ],
        out_shape=[
            jax.ShapeDtypeStruct((n, kc), jnp.float32),
            jax.ShapeDtypeStruct((n, kb), jnp.float32),
        ],
        scratch_shapes=[
            pltpu.VMEM((NBUF, BN, d), jnp.float32),
            pltpu.VMEM((NBUF, BN, kc), jnp.float32),
            pltpu.VMEM((NBUF, BN, kb), jnp.float32),
            pltpu.SemaphoreType.DMA((NBUF,)),
            pltpu.SemaphoreType.DMA((NBUF,)),
            pltpu.SemaphoreType.DMA((NBUF,)),
        ],
    )(x, W_cls, bc, W_box, bb)
    return (scores, deltas)
